# Initial kernel scaffold; baseline (speedup 1.0000x reference)
import functools

import jax
import jax.numpy as jnp
from jax import lax
from jax.experimental import pallas as pl
from jax.experimental.pallas import tpu as pltpu
from jax.experimental.pallas import tpu_sc as plsc

VOCAB = 1000000
EMBED_DIM = 64
BATCH = 4096
HIST = 50

N = BATCH * HIST
NC = 2
NS = 16
NW = NC * NS
PER_W = N // NW
CHUNK = 800
NCHUNK = PER_W // CHUNK


def _body(x1_hbm, x2_hbm, table_hbm, out1_hbm, out2_hbm,
          idx0, idx1, rows0, rows1, sem0, sem1):
    wid = lax.axis_index("s") * NC + lax.axis_index("c")
    base = wid * PER_W

    idx_bufs = (idx0, idx1)
    row_bufs = (rows0, rows1)
    sems = (sem0, sem1)

    sched = [(x1_hbm, out1_hbm, c) for c in range(NCHUNK)]
    sched += [(x2_hbm, out2_hbm, c) for c in range(NCHUNK)]

    handles = [None, None]
    for k, (src, dst, c) in enumerate(sched):
        b = k & 1
        off = base + c * CHUNK
        pltpu.sync_copy(src.at[pl.ds(off, CHUNK)], idx_bufs[b])
        handles[b] = pltpu.async_copy(table_hbm.at[idx_bufs[b]],
                                      row_bufs[b], sems[b])
        if k > 0:
            pb = 1 - b
            _, pdst, pc = sched[k - 1]
            handles[pb].wait()
            pltpu.sync_copy(row_bufs[pb],
                            pdst.at[pl.ds(base + pc * CHUNK, CHUNK)])
    lb = (len(sched) - 1) & 1
    _, ldst, lc = sched[-1]
    handles[lb].wait()
    pltpu.sync_copy(row_bufs[lb], ldst.at[pl.ds(base + lc * CHUNK, CHUNK)])


_sc_kernel = functools.partial(
    pl.kernel,
    out_type=(jax.ShapeDtypeStruct((N, EMBED_DIM), jnp.float32),
              jax.ShapeDtypeStruct((N, EMBED_DIM), jnp.float32)),
    mesh=plsc.VectorSubcoreMesh(core_axis_name="c", subcore_axis_name="s"),
    scratch_types=[
        pltpu.VMEM((CHUNK,), jnp.int32),
        pltpu.VMEM((CHUNK,), jnp.int32),
        pltpu.VMEM((CHUNK, EMBED_DIM), jnp.float32),
        pltpu.VMEM((CHUNK, EMBED_DIM), jnp.float32),
        pltpu.SemaphoreType.DMA,
        pltpu.SemaphoreType.DMA,
    ],
)(_body)


def kernel(x1, x2, table):
    f1 = x1.reshape(-1).astype(jnp.int32)
    f2 = x2.reshape(-1).astype(jnp.int32)
    o1, o2 = _sc_kernel(f1, f2, table)
    return (o1.reshape(BATCH, HIST, EMBED_DIM),
            o2.reshape(BATCH, HIST, EMBED_DIM))


# speedup vs baseline: 1.3014x; 1.3014x over previous
import functools

import jax
import jax.numpy as jnp
from jax import lax
from jax.experimental import pallas as pl
from jax.experimental.pallas import tpu as pltpu
from jax.experimental.pallas import tpu_sc as plsc

VOCAB = 1000000
EMBED_DIM = 64
BATCH = 4096
HIST = 50

N = BATCH * HIST
NC = 2
NS = 16
NW = NC * NS
PER_W = N // NW
CHUNK = 800
NCHUNK = PER_W // CHUNK


def _body(x1_hbm, x2_hbm, table_hbm, out1_hbm, out2_hbm,
          idx0, idx1, rows0, rows1, sem0, sem1):
    wid = lax.axis_index("s") * NC + lax.axis_index("c")
    base = wid * PER_W

    idx_bufs = (idx0, idx1)
    row_bufs = (rows0, rows1)
    sems = (sem0, sem1)

    sched = [(x1_hbm, out1_hbm, c) for c in range(NCHUNK)]
    sched += [(x2_hbm, out2_hbm, c) for c in range(NCHUNK)]

    handles = [None, None]
    for k, (src, dst, c) in enumerate(sched):
        b = k & 1
        off = base + c * CHUNK
        pltpu.sync_copy(src.at[pl.ds(off, CHUNK)], idx_bufs[b])
        handles[b] = pltpu.async_copy(table_hbm.at[idx_bufs[b]],
                                      row_bufs[b], sems[b])
        if k > 0:
            pb = 1 - b
            _, pdst, pc = sched[k - 1]
            handles[pb].wait()
            pltpu.sync_copy(row_bufs[pb],
                            pdst.at[pl.ds(base + pc * CHUNK, CHUNK)])
    lb = (len(sched) - 1) & 1
    _, ldst, lc = sched[-1]
    handles[lb].wait()
    pltpu.sync_copy(row_bufs[lb], ldst.at[pl.ds(base + lc * CHUNK, CHUNK)])


_sc_kernel = functools.partial(
    pl.kernel,
    out_type=(jax.ShapeDtypeStruct((N, EMBED_DIM), jnp.float32),
              jax.ShapeDtypeStruct((N, EMBED_DIM), jnp.float32)),
    mesh=plsc.VectorSubcoreMesh(core_axis_name="c", subcore_axis_name="s"),
    scratch_types=[
        pltpu.VMEM((CHUNK,), jnp.int32),
        pltpu.VMEM((CHUNK,), jnp.int32),
        pltpu.VMEM((CHUNK, EMBED_DIM), jnp.float32),
        pltpu.VMEM((CHUNK, EMBED_DIM), jnp.float32),
        pltpu.SemaphoreType.DMA,
        pltpu.SemaphoreType.DMA,
    ],
    compiler_params=pltpu.CompilerParams(use_tc_tiling_on_sc=False),
)(_body)


def kernel(x1, x2, table):
    f1 = x1.reshape(-1).astype(jnp.int32)
    f2 = x2.reshape(-1).astype(jnp.int32)
    o1, o2 = _sc_kernel(f1, f2, table)
    return (o1.reshape(BATCH, HIST, EMBED_DIM),
            o2.reshape(BATCH, HIST, EMBED_DIM))
